# Initial kernel scaffold; baseline (speedup 1.0000x reference)
import jax
import jax.numpy as jnp
from jax.experimental import pallas as pl


def kernel(x, edge_index, edge_attr, atom_W, atom_b, bond_W, bond_b, conv_W1, conv_b1, conv_W2, conv_b2, jk_W, jk_b):
    raise NotImplementedError("write your pallas kernel here")


# speedup vs baseline: 2.0295x; 2.0295x over previous
import functools

import jax
import jax.numpy as jnp
from jax import lax
from jax.experimental import pallas as pl
from jax.experimental.pallas import tpu as pltpu
from jax.experimental.pallas import tpu_sc as plsc

F32 = jnp.float32

NUM_CORES = 2
NUM_SUBCORES = 16
LANES = 16


def _enc_body(x_ref, w_ref, b_ref, o0_ref, o1_ref):
    h = jnp.dot(x_ref[...], w_ref[...], preferred_element_type=F32) + b_ref[...]
    half = h.shape[1] // 2
    o0_ref[...] = h[:, :half]
    o1_ref[...] = h[:, half:]


def _encode_halves(x, W, b, blk):
    n, d = x.shape
    h = W.shape[1]
    assert n % blk == 0
    return pl.pallas_call(
        _enc_body,
        grid=(n // blk,),
        in_specs=[
            pl.BlockSpec((blk, d), lambda i: (i, 0)),
            pl.BlockSpec((d, h), lambda i: (0, 0)),
            pl.BlockSpec((1, h), lambda i: (0, 0)),
        ],
        out_specs=[
            pl.BlockSpec((blk, h // 2), lambda i: (i, 0)),
            pl.BlockSpec((blk, h // 2), lambda i: (i, 0)),
        ],
        out_shape=[
            jax.ShapeDtypeStruct((n, h // 2), F32),
            jax.ShapeDtypeStruct((n, h // 2), F32),
        ],
    )(x, W, b.reshape(1, h))


def _mlp_body(h0, h1, a0, a1, w1, b1, w2, b2, o0, o1):
    z0 = h0[...] + a0[...]
    z1 = h1[...] + a1[...]
    t = jnp.dot(z0, w1[0], preferred_element_type=F32)
    t = t + jnp.dot(z1, w1[1], preferred_element_type=F32)
    t = jnp.maximum(t + b1[...], 0.0)
    u = jnp.dot(t, w2[...], preferred_element_type=F32) + b2[...]
    u = jnp.maximum(u, 0.0)
    half = u.shape[1] // 2
    o0[...] = u[:, :half]
    o1[...] = u[:, half:]


def _mlp_halves(h0, h1, a0, a1, W1, b1, W2, b2, blk):
    n, half = h0.shape
    hdim = W1.shape[1]
    assert n % blk == 0
    w1r = W1.reshape(2, half, hdim)
    return pl.pallas_call(
        _mlp_body,
        grid=(n // blk,),
        in_specs=[
            pl.BlockSpec((blk, half), lambda i: (i, 0)),
            pl.BlockSpec((blk, half), lambda i: (i, 0)),
            pl.BlockSpec((blk, half), lambda i: (i, 0)),
            pl.BlockSpec((blk, half), lambda i: (i, 0)),
            pl.BlockSpec((2, half, hdim), lambda i: (0, 0, 0)),
            pl.BlockSpec((1, hdim), lambda i: (0, 0)),
            pl.BlockSpec((hdim, hdim), lambda i: (0, 0)),
            pl.BlockSpec((1, hdim), lambda i: (0, 0)),
        ],
        out_specs=[
            pl.BlockSpec((blk, half), lambda i: (i, 0)),
            pl.BlockSpec((blk, half), lambda i: (i, 0)),
        ],
        out_shape=[
            jax.ShapeDtypeStruct((n, half), F32),
            jax.ShapeDtypeStruct((n, half), F32),
        ],
    )(h0, h1, a0, a1, w1r, b1.reshape(1, hdim), W2, b2.reshape(1, hdim))


def _jk_body(*refs):
    xs = refs[:-3]
    w_ref, b_ref, o_ref = refs[-3:]
    acc = jnp.broadcast_to(b_ref[...], o_ref.shape).astype(F32)
    for i, x in enumerate(xs):
        acc = acc + jnp.dot(x[...], w_ref[i], preferred_element_type=F32)
    o_ref[...] = acc


def _jk_project(xs_halves, jk_W, jk_b, blk):
    n, half = xs_halves[0].shape
    k = len(xs_halves)
    dout = jk_W.shape[1]
    assert n % blk == 0
    wr = jk_W.reshape(k, half, dout)
    in_specs = [pl.BlockSpec((blk, half), lambda i: (i, 0)) for _ in range(k)]
    in_specs.append(pl.BlockSpec((k, half, dout), lambda i: (0, 0, 0)))
    in_specs.append(pl.BlockSpec((1, dout), lambda i: (0, 0)))
    return pl.pallas_call(
        _jk_body,
        grid=(n // blk,),
        in_specs=in_specs,
        out_specs=pl.BlockSpec((blk, dout), lambda i: (i, 0)),
        out_shape=jax.ShapeDtypeStruct((n, dout), F32),
    )(*xs_halves, wr, jk_b.reshape(1, dout))


def _make_edge_fn(n_nodes, n_edges, half):
    ept = n_edges // NUM_SUBCORES
    assert n_edges % NUM_SUBCORES == 0
    ch = 128
    while ept % ch != 0:
        ch -= 8
    n_chunks = ept // ch
    rows_per_tile = n_nodes // NUM_SUBCORES
    assert n_nodes % NUM_SUBCORES == 0

    mesh = plsc.VectorSubcoreMesh(core_axis_name="c", subcore_axis_name="s")

    @functools.partial(
        pl.kernel,
        mesh=mesh,
        out_type=[
            jax.ShapeDtypeStruct((n_nodes, half), F32),
            jax.ShapeDtypeStruct((n_nodes, half), F32),
        ],
        scratch_types=[
            pltpu.VMEM((ch,), jnp.int32),
            pltpu.VMEM((ch,), jnp.int32),
            pltpu.VMEM((ch, half), F32),
            pltpu.VMEM((ch, half), F32),
            pltpu.VMEM_SHARED((n_nodes, half), F32),
            pltpu.SemaphoreType.DMA,
        ],
    )
    def edge_fn(h0, h1, e0, e1, src, dst, zeros, out0, out1,
                src_v, dst_v, rows_v, e_v, agg_sh, sem):
        c = lax.axis_index("c")
        s = lax.axis_index("s")
        row_base = s * rows_per_tile

        pltpu.sync_copy(zeros.at[pl.ds(row_base, rows_per_tile)],
                        agg_sh.at[pl.ds(row_base, rows_per_tile)])
        plsc.subcore_barrier()

        def do_half(h_ref, e_ref, out_ref):
            tile_base = s * ept

            def chunk(g, carry):
                eb = tile_base + g * ch
                pltpu.sync_copy(src.at[pl.ds(eb, ch)], src_v)
                pltpu.sync_copy(dst.at[pl.ds(eb, ch)], dst_v)
                pltpu.sync_copy(e_ref.at[pl.ds(eb, ch)], e_v)
                pltpu.async_copy(h_ref.at[src_v], rows_v, sem).wait()

                def row(i, rcarry):
                    for k in range(half // LANES):
                        sl = pl.ds(k * LANES, LANES)
                        v = rows_v[i, sl] + e_v[i, sl]
                        rows_v[i, sl] = jnp.maximum(v, 0.0)
                    return rcarry

                lax.fori_loop(0, ch, row, 0)
                pltpu.sync_copy(rows_v, agg_sh.at[dst_v], add=True)
                return carry

            lax.fori_loop(0, n_chunks, chunk, 0)
            plsc.subcore_barrier()
            pltpu.sync_copy(agg_sh.at[pl.ds(row_base, rows_per_tile)],
                            out_ref.at[pl.ds(row_base, rows_per_tile)])

        @pl.when(c == 0)
        def _():
            do_half(h0, e0, out0)

        @pl.when(c == 1)
        def _():
            do_half(h1, e1, out1)

    return edge_fn


def kernel(x, edge_index, edge_attr, atom_W, atom_b, bond_W, bond_b,
           conv_W1, conv_b1, conv_W2, conv_b2, jk_W, jk_b):
    n_nodes = x.shape[0]
    n_edges = edge_index.shape[1]
    hdim = atom_W.shape[1]
    half = hdim // 2
    n_layers = conv_W1.shape[0]

    n_pad = ((n_nodes + NUM_SUBCORES * 8 * 8 - 1)
             // (NUM_SUBCORES * 8 * 8)) * (NUM_SUBCORES * 8 * 8)
    x = jnp.pad(x, ((0, n_pad - n_nodes), (0, 0)))

    src = edge_index[0]
    dst = edge_index[1]
    zeros = jnp.zeros((n_pad, half), dtype=F32)

    node_blk = n_pad // 16
    edge_blk = 2000

    h0, h1 = _encode_halves(x, atom_W, atom_b, node_blk)
    e0, e1 = _encode_halves(edge_attr, bond_W, bond_b, edge_blk)

    edge_fn = _make_edge_fn(n_pad, n_edges, half)

    xs = [h0, h1]
    for l in range(n_layers):
        a0, a1 = edge_fn(h0, h1, e0, e1, src, dst, zeros)
        h0, h1 = _mlp_halves(h0, h1, a0, a1, conv_W1[l], conv_b1[l],
                             conv_W2[l], conv_b2[l], node_blk)
        xs.extend([h0, h1])

    return _jk_project(xs, jk_W, jk_b, node_blk)[:n_nodes]


# speedup vs baseline: 4.1833x; 2.0612x over previous
import functools

import jax
import jax.numpy as jnp
from jax import lax
from jax.experimental import pallas as pl
from jax.experimental.pallas import tpu as pltpu
from jax.experimental.pallas import tpu_sc as plsc

F32 = jnp.float32

NUM_CORES = 2
NUM_SUBCORES = 16
LANES = 16


def _enc_body(x_ref, w_ref, b_ref, o0_ref, o1_ref):
    h = jnp.dot(x_ref[...], w_ref[...], preferred_element_type=F32) + b_ref[...]
    half = h.shape[1] // 2
    o0_ref[...] = h[:, :half]
    o1_ref[...] = h[:, half:]


def _encode_halves(x, W, b, blk):
    n, d = x.shape
    h = W.shape[1]
    assert n % blk == 0
    return pl.pallas_call(
        _enc_body,
        grid=(n // blk,),
        in_specs=[
            pl.BlockSpec((blk, d), lambda i: (i, 0)),
            pl.BlockSpec((d, h), lambda i: (0, 0)),
            pl.BlockSpec((1, h), lambda i: (0, 0)),
        ],
        out_specs=[
            pl.BlockSpec((blk, h // 2), lambda i: (i, 0)),
            pl.BlockSpec((blk, h // 2), lambda i: (i, 0)),
        ],
        out_shape=[
            jax.ShapeDtypeStruct((n, h // 2), F32),
            jax.ShapeDtypeStruct((n, h // 2), F32),
        ],
    )(x, W, b.reshape(1, h))


def _mlp_body(h0, h1, a0, a1, w1, b1, w2, b2, o0, o1):
    z0 = h0[...] + a0[...]
    z1 = h1[...] + a1[...]
    t = jnp.dot(z0, w1[0], preferred_element_type=F32)
    t = t + jnp.dot(z1, w1[1], preferred_element_type=F32)
    t = jnp.maximum(t + b1[...], 0.0)
    u = jnp.dot(t, w2[...], preferred_element_type=F32) + b2[...]
    u = jnp.maximum(u, 0.0)
    half = u.shape[1] // 2
    o0[...] = u[:, :half]
    o1[...] = u[:, half:]


def _mlp_halves(h0, h1, a0, a1, W1, b1, W2, b2, blk):
    n, half = h0.shape
    hdim = W1.shape[1]
    assert n % blk == 0
    w1r = W1.reshape(2, half, hdim)
    return pl.pallas_call(
        _mlp_body,
        grid=(n // blk,),
        in_specs=[
            pl.BlockSpec((blk, half), lambda i: (i, 0)),
            pl.BlockSpec((blk, half), lambda i: (i, 0)),
            pl.BlockSpec((blk, half), lambda i: (i, 0)),
            pl.BlockSpec((blk, half), lambda i: (i, 0)),
            pl.BlockSpec((2, half, hdim), lambda i: (0, 0, 0)),
            pl.BlockSpec((1, hdim), lambda i: (0, 0)),
            pl.BlockSpec((hdim, hdim), lambda i: (0, 0)),
            pl.BlockSpec((1, hdim), lambda i: (0, 0)),
        ],
        out_specs=[
            pl.BlockSpec((blk, half), lambda i: (i, 0)),
            pl.BlockSpec((blk, half), lambda i: (i, 0)),
        ],
        out_shape=[
            jax.ShapeDtypeStruct((n, half), F32),
            jax.ShapeDtypeStruct((n, half), F32),
        ],
    )(h0, h1, a0, a1, w1r, b1.reshape(1, hdim), W2, b2.reshape(1, hdim))


def _jk_body(*refs):
    xs = refs[:-3]
    w_ref, b_ref, o_ref = refs[-3:]
    acc = jnp.broadcast_to(b_ref[...], o_ref.shape).astype(F32)
    for i, x in enumerate(xs):
        acc = acc + jnp.dot(x[...], w_ref[i], preferred_element_type=F32)
    o_ref[...] = acc


def _jk_project(xs_halves, jk_W, jk_b, blk):
    n, half = xs_halves[0].shape
    k = len(xs_halves)
    dout = jk_W.shape[1]
    assert n % blk == 0
    wr = jk_W.reshape(k, half, dout)
    in_specs = [pl.BlockSpec((blk, half), lambda i: (i, 0)) for _ in range(k)]
    in_specs.append(pl.BlockSpec((k, half, dout), lambda i: (0, 0, 0)))
    in_specs.append(pl.BlockSpec((1, dout), lambda i: (0, 0)))
    return pl.pallas_call(
        _jk_body,
        grid=(n // blk,),
        in_specs=in_specs,
        out_specs=pl.BlockSpec((blk, dout), lambda i: (i, 0)),
        out_shape=jax.ShapeDtypeStruct((n, dout), F32),
    )(*xs_halves, wr, jk_b.reshape(1, dout))


def _make_edge_fn(n_nodes, n_edges, half):
    ept = n_edges // NUM_SUBCORES
    assert n_edges % NUM_SUBCORES == 0
    ch = 128
    while ept % ch != 0:
        ch -= 8
    n_chunks = ept // ch
    assert n_chunks % 2 == 0
    rows_per_tile = n_nodes // NUM_SUBCORES
    assert n_nodes % NUM_SUBCORES == 0

    mesh = plsc.VectorSubcoreMesh(core_axis_name="c", subcore_axis_name="s")

    @functools.partial(
        pl.kernel,
        mesh=mesh,
        out_type=[
            jax.ShapeDtypeStruct((n_nodes, half), F32),
            jax.ShapeDtypeStruct((n_nodes, half), F32),
        ],
        scratch_types=[
            pltpu.VMEM((2, 2, ch), jnp.int32),
            pltpu.VMEM((2, ch, half), F32),
            pltpu.VMEM((2, ch, half), F32),
            pltpu.VMEM_SHARED((n_nodes, half), F32),
            pltpu.SemaphoreType.DMA,
            pltpu.SemaphoreType.DMA,
            pltpu.SemaphoreType.DMA,
            pltpu.SemaphoreType.DMA,
        ],
    )
    def edge_fn(h0, h1, e0, e1, idx4, zeros, out0, out1,
                idx_v, rows_v, e_v, agg_sh, sg0, sg1, se0, se1):
        c = lax.axis_index("c")
        s = lax.axis_index("s")
        row_base = s * rows_per_tile
        sgs = (sg0, sg1)
        ses = (se0, se1)

        pltpu.sync_copy(zeros.at[pl.ds(row_base, rows_per_tile)],
                        agg_sh.at[pl.ds(row_base, rows_per_tile)])
        plsc.subcore_barrier()

        def do_half(h_ref, e_ref, out_ref):
            tile_base = s * ept

            def start(g, p):
                pltpu.sync_copy(idx4.at[s, g], idx_v.at[p])
                pltpu.async_copy(h_ref.at[idx_v.at[p, 0]], rows_v.at[p],
                                 sgs[p])
                pltpu.async_copy(e_ref.at[pl.ds(tile_base + g * ch, ch)],
                                 e_v.at[p], ses[p])

            def wait(p):
                pltpu.make_async_copy(h_ref.at[idx_v.at[p, 0]], rows_v.at[p],
                                      sgs[p]).wait()
                pltpu.make_async_copy(e_ref.at[pl.ds(tile_base, ch)],
                                      e_v.at[p], ses[p]).wait()

            start(0, 0)
            start(1, 1)

            def pair(gp, carry):
                g = gp * 2
                for p in (0, 1):
                    gg = g + p
                    wait(p)

                    @plsc.parallel_loop(0, ch, 1, unroll=2)
                    def _row(i):
                        for k in range(half // LANES):
                            sl = pl.ds(k * LANES, LANES)
                            v = rows_v[p, i, sl] + e_v[p, i, sl]
                            rows_v[p, i, sl] = jnp.maximum(v, 0.0)

                    pltpu.sync_copy(rows_v.at[p], agg_sh.at[idx_v.at[p, 1]],
                                    add=True)

                    @pl.when(gg + 2 < n_chunks)
                    def _():
                        start(gg + 2, p)
                return carry

            lax.fori_loop(0, n_chunks // 2, pair, 0)
            plsc.subcore_barrier()
            pltpu.sync_copy(agg_sh.at[pl.ds(row_base, rows_per_tile)],
                            out_ref.at[pl.ds(row_base, rows_per_tile)])

        @pl.when(c == 0)
        def _():
            do_half(h0, e0, out0)

        @pl.when(c == 1)
        def _():
            do_half(h1, e1, out1)

    return edge_fn


def kernel(x, edge_index, edge_attr, atom_W, atom_b, bond_W, bond_b,
           conv_W1, conv_b1, conv_W2, conv_b2, jk_W, jk_b):
    n_nodes = x.shape[0]
    n_edges = edge_index.shape[1]
    hdim = atom_W.shape[1]
    half = hdim // 2
    n_layers = conv_W1.shape[0]

    n_pad = ((n_nodes + NUM_SUBCORES * 8 * 8 - 1)
             // (NUM_SUBCORES * 8 * 8)) * (NUM_SUBCORES * 8 * 8)
    x = jnp.pad(x, ((0, n_pad - n_nodes), (0, 0)))

    ept = n_edges // NUM_SUBCORES
    ch = 128
    while ept % ch != 0:
        ch -= 8
    idx4 = edge_index.reshape(2, NUM_SUBCORES, ept // ch, ch).transpose(1, 2, 0, 3)
    zeros = jnp.zeros((n_pad, half), dtype=F32)

    node_blk = n_pad // 16
    edge_blk = 2000

    h0, h1 = _encode_halves(x, atom_W, atom_b, node_blk)
    e0, e1 = _encode_halves(edge_attr, bond_W, bond_b, edge_blk)

    edge_fn = _make_edge_fn(n_pad, n_edges, half)

    xs = [h0, h1]
    for l in range(n_layers):
        a0, a1 = edge_fn(h0, h1, e0, e1, idx4, zeros)
        h0, h1 = _mlp_halves(h0, h1, a0, a1, conv_W1[l], conv_b1[l],
                             conv_W2[l], conv_b2[l], node_blk)
        xs.extend([h0, h1])

    return _jk_project(xs, jk_W, jk_b, node_blk)[:n_nodes]
